# baseline (speedup 1.0000x reference)
import functools

import jax
import jax.numpy as jnp
from jax import lax
from jax.experimental import pallas as pl
from jax.experimental.pallas import tpu as pltpu
from jax.experimental.pallas import tpu_sc as plsc

BS = 8
SEQ_LEN = 256
ENC_IN = 32
D_MODEL = 128
N_FREQ_MODES = 32
N_BINS = SEQ_LEN // 2 + 1
FPAD = SEQ_LEN

_HI = jax.lax.Precision.HIGHEST


def _dot(a, b):
    return jax.lax.dot(a, b, precision=_HI)


def _main_body(x_ref, mask_ref, cos_ref, sin_ref, wtime_ref, btime_ref,
               wfreq_ref, bfreq_ref, wstat_ref, bstat_ref, wtemp_ref,
               btemp_ref, wfhe_ref, bfhe_ref, wcross_ref, bcross_ref,
               tn_ref, fn_ref, sn_ref, the_ref, fhe_ref, cross_ref):
    f32 = jnp.float32
    x = x_ref[0]
    mask = mask_ref[0]

    wt = wtime_ref[...].reshape(1, 1, D_MODEL)
    bt = btime_ref[...].reshape(1, 1, D_MODEL)
    tn = jnp.maximum(x[:, :, None] * wt + bt, 0.0)
    tn_ref[0] = tn
    tn_mean_c = jnp.mean(tn, axis=1)
    the_ref[0] = _dot(tn_mean_c, wtemp_ref[...]) + btemp_ref[...]
    tmean = jnp.mean(tn_mean_c, axis=0, keepdims=True)

    re = _dot(cos_ref[...], x)
    im = -_dot(sin_ref[...], x)
    mag = jnp.sqrt(re * re + im * im)
    mag_col = jnp.mean(mag, axis=1, keepdims=True)
    sub2 = lax.broadcasted_iota(jnp.int32, (FPAD, 1), 0)
    mag_col = jnp.where(sub2 < N_BINS, mag_col, -1.0)

    eyeF = (lax.broadcasted_iota(jnp.int32, (FPAD, FPAD), 0)
            == lax.broadcasted_iota(jnp.int32, (FPAD, FPAD), 1)).astype(f32)
    ones_row = jnp.ones((1, FPAD), f32)
    mag_row = _dot(ones_row, eyeF * mag_col)

    lane_i = lax.broadcasted_iota(jnp.int32, (FPAD, FPAD), 1)
    sub_i = lax.broadcasted_iota(jnp.int32, (FPAD, FPAD), 0)
    beats = (mag_row > mag_col) | ((mag_row == mag_col) & (lane_i < sub_i))
    rank_col = jnp.sum(beats.astype(f32), axis=1, keepdims=True)
    rank_row = _dot(ones_row, eyeF * rank_col)
    r_iota = lax.broadcasted_iota(jnp.int32, (N_FREQ_MODES, FPAD), 0)
    onehot = (rank_row.astype(jnp.int32) == r_iota).astype(f32)
    sel_re = _dot(onehot, re)
    sel_im = _dot(onehot, im)

    wf = wfreq_ref[...]
    wf0 = wf[0:1, :].reshape(1, 1, D_MODEL)
    wf1 = wf[1:2, :].reshape(1, 1, D_MODEL)
    bf = bfreq_ref[...].reshape(1, 1, D_MODEL)
    fn = jnp.maximum(sel_re[:, :, None] * wf0 + sel_im[:, :, None] * wf1 + bf,
                     0.0)
    fn_ref[0] = fn
    fn_mean = jnp.mean(fn, axis=1)
    fhe_ref[0] = _dot(fn_mean, wfhe_ref[...]) + bfhe_ref[...]
    fmean = jnp.mean(fn_mean, axis=0, keepdims=True)

    mx = x * mask
    n_obs = jnp.maximum(jnp.sum(mask, axis=0, keepdims=True), 1.0)
    mean_r = jnp.sum(mx, axis=0, keepdims=True) / n_obs
    var_r = jnp.sum(((mx - mean_r) ** 2) * mask, axis=0, keepdims=True) / n_obs
    tcol = lax.broadcasted_iota(jnp.int32, (SEQ_LEN, ENC_IN), 0).astype(f32)
    tbar = (SEQ_LEN - 1) / 2.0
    trend_r = jnp.sum(mx * tcol * mask, axis=0, keepdims=True) / n_obs \
        - mean_r * tbar

    eyeC = (lax.broadcasted_iota(jnp.int32, (ENC_IN, ENC_IN), 0)
            == lax.broadcasted_iota(jnp.int32, (ENC_IN, ENC_IN), 1)).astype(f32)
    ones_colC = jnp.ones((ENC_IN, 1), f32)

    def row_to_col(r):
        return _dot(eyeC * r, ones_colC)

    sfT = jnp.concatenate(
        [row_to_col(mean_r), row_to_col(var_r), row_to_col(trend_r)], axis=1)
    stat_base = jnp.maximum(_dot(sfT, wstat_ref[...]) + bstat_ref[...], 0.0)
    sn_ref[0] = jnp.broadcast_to(stat_base[None], (3, ENC_IN, D_MODEL))
    smean = jnp.mean(stat_base, axis=0, keepdims=True)

    cross_cat = jnp.concatenate([tmean, fmean, smean], axis=0)
    cross_ref[0] = _dot(cross_cat, wcross_ref[...]) + bcross_ref[...]


_NC, _NS = 2, 16
_NW = _NC * _NS
_ROWS = BS * SEQ_LEN
_RPW = _ROWS // _NW
_NCOL = SEQ_LEN * ENC_IN


_NBUF = 4
_PACE = 400


def _inc_sc_body(mask_hbm, out_hbm, maskv, z0, z1, z2, z3,
                 sem0, sem1, sem2, sem3, semm):
    f32 = jnp.float32
    wid = lax.axis_index("s") * _NC + lax.axis_index("c")
    base = wid * _RPW
    stage = pltpu.make_async_copy(
        mask_hbm.at[pl.ds(2 * base, 2 * _RPW)], maskv, semm)
    stage.start()

    zeros16 = jnp.zeros((16,), f32)
    bufs = (z0, z1, z2, z3)
    sems = (sem0, sem1, sem2, sem3)

    def zb(i, c):
        for zk in bufs:
            for j in range(8):
                zk[pl.ds(i * 128 + j * 16, 16)] = zeros16
        return c

    lax.fori_loop(0, _NCOL // 128, zb, 0)
    stage.wait()

    def body(i, c):
        for k in range(_NBUF):
            r = _NBUF * i + k
            g = base + r
            off = lax.rem(g, SEQ_LEN) * ENC_IN
            zk = bufs[k]
            sk = sems[k]

            @pl.when(i > 0)
            def _():
                pltpu.make_async_copy(zk, out_hbm.at[g - _NBUF], sk).wait()
                poff = lax.rem(g - _NBUF, SEQ_LEN) * ENC_IN
                zk[pl.ds(poff, 16)] = zeros16
                zk[pl.ds(poff + 16, 16)] = zeros16

            zk[pl.ds(off, 16)] = maskv[2 * r]
            zk[pl.ds(off + 16, 16)] = maskv[2 * r + 1]
            pltpu.async_copy(zk, out_hbm.at[g], sk)
            pl.delay(_PACE)
        return c

    lax.fori_loop(0, _RPW // _NBUF, body, 0)

    for k in range(_NBUF):
        g = base + _RPW - _NBUF + k
        pltpu.make_async_copy(bufs[k], out_hbm.at[g], sems[k]).wait()


_TBLK = 32


def _inc_body(mask_ref, out_ref):
    tb = pl.program_id(1)
    m = mask_ref[0]
    ncol = SEQ_LEN * ENC_IN
    jB = lax.broadcasted_iota(jnp.int32, (ENC_IN, ncol), 1)
    cB = lax.broadcasted_iota(jnp.int32, (ENC_IN, ncol), 0)
    B = (jB - (jB // ENC_IN) * ENC_IN == cB).astype(jnp.float32)
    rep = _dot(m, B)
    j = lax.broadcasted_iota(jnp.int32, (_TBLK, ncol), 1)
    tg = lax.broadcasted_iota(jnp.int32, (_TBLK, ncol), 0) + tb * _TBLK
    lo = tg * ENC_IN
    keep = (j >= lo) & (j < lo + ENC_IN)
    out_ref[0] = jnp.where(keep, rep, 0.0)


@functools.partial(jax.jit, static_argnames=())
def kernel(x, mask, W_time, b_time, W_freq, b_freq, W_stat, b_stat,
           W_temp, b_temp, W_fhe, b_fhe, W_cross, b_cross):
    f32 = jnp.float32
    f_idx = jnp.arange(FPAD, dtype=f32)[:, None]
    t_idx = jnp.arange(SEQ_LEN, dtype=f32)[None, :]
    ang = (2.0 * jnp.pi / SEQ_LEN) * f_idx * t_idx
    live = (f_idx < N_BINS).astype(f32)
    cos_m = jnp.cos(ang) * live
    sin_m = jnp.sin(ang) * live

    b_time2 = b_time.reshape(1, D_MODEL)
    b_freq2 = b_freq.reshape(1, D_MODEL)
    b_stat2 = b_stat.reshape(1, D_MODEL)
    b_temp2 = b_temp.reshape(1, D_MODEL)
    b_fhe2 = b_fhe.reshape(1, D_MODEL)
    b_cross2 = b_cross.reshape(1, D_MODEL)

    full = lambda shape: pl.BlockSpec(shape, lambda b: (0,) * len(shape))
    per_b2 = pl.BlockSpec((1, SEQ_LEN, ENC_IN), lambda b: (b, 0, 0))

    inc_flat = pl.kernel(
        _inc_sc_body,
        out_type=jax.ShapeDtypeStruct((_ROWS, _NCOL), f32),
        mesh=plsc.VectorSubcoreMesh(core_axis_name="c", subcore_axis_name="s"),
        scratch_types=[
            pltpu.VMEM((2 * _RPW, 16), f32),
            pltpu.VMEM((_NCOL,), f32),
            pltpu.VMEM((_NCOL,), f32),
            pltpu.VMEM((_NCOL,), f32),
            pltpu.VMEM((_NCOL,), f32),
            pltpu.SemaphoreType.DMA,
            pltpu.SemaphoreType.DMA,
            pltpu.SemaphoreType.DMA,
            pltpu.SemaphoreType.DMA,
            pltpu.SemaphoreType.DMA,
        ],
    )(mask.reshape(2 * _ROWS, 16))
    incidence = inc_flat.reshape(BS, SEQ_LEN, SEQ_LEN * ENC_IN)

    tn, fn, sn, the, fhe, cross = pl.pallas_call(
        _main_body,
        grid=(BS,),
        in_specs=[
            per_b2, per_b2,
            full((FPAD, SEQ_LEN)), full((FPAD, SEQ_LEN)),
            full((1, D_MODEL)), full((1, D_MODEL)),
            full((2, D_MODEL)), full((1, D_MODEL)),
            full((3, D_MODEL)), full((1, D_MODEL)),
            full((D_MODEL, D_MODEL)), full((1, D_MODEL)),
            full((D_MODEL, D_MODEL)), full((1, D_MODEL)),
            full((D_MODEL, D_MODEL)), full((1, D_MODEL)),
        ],
        out_specs=[
            pl.BlockSpec((1, SEQ_LEN, ENC_IN, D_MODEL), lambda b: (b, 0, 0, 0)),
            pl.BlockSpec((1, N_FREQ_MODES, ENC_IN, D_MODEL),
                         lambda b: (b, 0, 0, 0)),
            pl.BlockSpec((1, 3, ENC_IN, D_MODEL), lambda b: (b, 0, 0, 0)),
            pl.BlockSpec((1, SEQ_LEN, D_MODEL), lambda b: (b, 0, 0)),
            pl.BlockSpec((1, N_FREQ_MODES, D_MODEL), lambda b: (b, 0, 0)),
            pl.BlockSpec((1, 3, D_MODEL), lambda b: (b, 0, 0)),
        ],
        out_shape=[
            jax.ShapeDtypeStruct((BS, SEQ_LEN, ENC_IN, D_MODEL), f32),
            jax.ShapeDtypeStruct((BS, N_FREQ_MODES, ENC_IN, D_MODEL), f32),
            jax.ShapeDtypeStruct((BS, 3, ENC_IN, D_MODEL), f32),
            jax.ShapeDtypeStruct((BS, SEQ_LEN, D_MODEL), f32),
            jax.ShapeDtypeStruct((BS, N_FREQ_MODES, D_MODEL), f32),
            jax.ShapeDtypeStruct((BS, 3, D_MODEL), f32),
        ],
    )(x, mask, cos_m, sin_m, W_time, b_time2, W_freq, b_freq2, W_stat,
      b_stat2, W_temp, b_temp2, W_fhe, b_fhe2, W_cross, b_cross2)

    return (tn, fn, sn, the, fhe, cross, incidence, mask)


# speedup vs baseline: 2.0452x; 2.0452x over previous
import functools

import jax
import jax.numpy as jnp
from jax import lax
from jax.experimental import pallas as pl
from jax.experimental.pallas import tpu as pltpu
from jax.experimental.pallas import tpu_sc as plsc

BS = 8
SEQ_LEN = 256
ENC_IN = 32
D_MODEL = 128
N_FREQ_MODES = 32
N_BINS = SEQ_LEN // 2 + 1
FPAD = SEQ_LEN

_HI = jax.lax.Precision.HIGHEST


def _dot(a, b):
    return jax.lax.dot(a, b, precision=_HI)


def _main_body(x_ref, mask_ref, cos_ref, sin_ref, wtime_ref, btime_ref,
               wfreq_ref, bfreq_ref, wstat_ref, bstat_ref, wtemp_ref,
               btemp_ref, wfhe_ref, bfhe_ref, wcross_ref, bcross_ref,
               tn_ref, fn_ref, sn_ref, the_ref, fhe_ref, cross_ref):
    f32 = jnp.float32
    x = x_ref[0]
    mask = mask_ref[0]

    wt = wtime_ref[...].reshape(1, 1, D_MODEL)
    bt = btime_ref[...].reshape(1, 1, D_MODEL)
    tn = jnp.maximum(x[:, :, None] * wt + bt, 0.0)
    tn_ref[0] = tn
    tn_mean_c = jnp.mean(tn, axis=1)
    the_ref[0] = _dot(tn_mean_c, wtemp_ref[...]) + btemp_ref[...]
    tmean = jnp.mean(tn_mean_c, axis=0, keepdims=True)

    re = _dot(cos_ref[...], x)
    im = -_dot(sin_ref[...], x)
    mag = jnp.sqrt(re * re + im * im)
    mag_col = jnp.mean(mag, axis=1, keepdims=True)
    sub2 = lax.broadcasted_iota(jnp.int32, (FPAD, 1), 0)
    mag_col = jnp.where(sub2 < N_BINS, mag_col, -1.0)

    eyeF = (lax.broadcasted_iota(jnp.int32, (FPAD, FPAD), 0)
            == lax.broadcasted_iota(jnp.int32, (FPAD, FPAD), 1)).astype(f32)
    ones_row = jnp.ones((1, FPAD), f32)
    mag_row = _dot(ones_row, eyeF * mag_col)

    lane_i = lax.broadcasted_iota(jnp.int32, (FPAD, FPAD), 1)
    sub_i = lax.broadcasted_iota(jnp.int32, (FPAD, FPAD), 0)
    beats = (mag_row > mag_col) | ((mag_row == mag_col) & (lane_i < sub_i))
    rank_col = jnp.sum(beats.astype(f32), axis=1, keepdims=True)
    rank_row = _dot(ones_row, eyeF * rank_col)
    r_iota = lax.broadcasted_iota(jnp.int32, (N_FREQ_MODES, FPAD), 0)
    onehot = (rank_row.astype(jnp.int32) == r_iota).astype(f32)
    sel_re = _dot(onehot, re)
    sel_im = _dot(onehot, im)

    wf = wfreq_ref[...]
    wf0 = wf[0:1, :].reshape(1, 1, D_MODEL)
    wf1 = wf[1:2, :].reshape(1, 1, D_MODEL)
    bf = bfreq_ref[...].reshape(1, 1, D_MODEL)
    fn = jnp.maximum(sel_re[:, :, None] * wf0 + sel_im[:, :, None] * wf1 + bf,
                     0.0)
    fn_ref[0] = fn
    fn_mean = jnp.mean(fn, axis=1)
    fhe_ref[0] = _dot(fn_mean, wfhe_ref[...]) + bfhe_ref[...]
    fmean = jnp.mean(fn_mean, axis=0, keepdims=True)

    mx = x * mask
    n_obs = jnp.maximum(jnp.sum(mask, axis=0, keepdims=True), 1.0)
    mean_r = jnp.sum(mx, axis=0, keepdims=True) / n_obs
    var_r = jnp.sum(((mx - mean_r) ** 2) * mask, axis=0, keepdims=True) / n_obs
    tcol = lax.broadcasted_iota(jnp.int32, (SEQ_LEN, ENC_IN), 0).astype(f32)
    tbar = (SEQ_LEN - 1) / 2.0
    trend_r = jnp.sum(mx * tcol * mask, axis=0, keepdims=True) / n_obs \
        - mean_r * tbar

    eyeC = (lax.broadcasted_iota(jnp.int32, (ENC_IN, ENC_IN), 0)
            == lax.broadcasted_iota(jnp.int32, (ENC_IN, ENC_IN), 1)).astype(f32)
    ones_colC = jnp.ones((ENC_IN, 1), f32)

    def row_to_col(r):
        return _dot(eyeC * r, ones_colC)

    sfT = jnp.concatenate(
        [row_to_col(mean_r), row_to_col(var_r), row_to_col(trend_r)], axis=1)
    stat_base = jnp.maximum(_dot(sfT, wstat_ref[...]) + bstat_ref[...], 0.0)
    sn_ref[0] = jnp.broadcast_to(stat_base[None], (3, ENC_IN, D_MODEL))
    smean = jnp.mean(stat_base, axis=0, keepdims=True)

    cross_cat = jnp.concatenate([tmean, fmean, smean], axis=0)
    cross_ref[0] = _dot(cross_cat, wcross_ref[...]) + bcross_ref[...]


_NC, _NS = 2, 16
_NW = _NC * _NS
_ROWS = BS * SEQ_LEN
_RPW = _ROWS // _NW
_NCOL = SEQ_LEN * ENC_IN


def _inc_sc_body(mask_hbm, out_hbm, maskv, z0, z1, sem0, sem1):
    f32 = jnp.float32
    wid = lax.axis_index("s") * _NC + lax.axis_index("c")
    base = wid * _RPW
    pltpu.sync_copy(mask_hbm.at[pl.ds(2 * base, 2 * _RPW)], maskv)

    zeros16 = jnp.zeros((16,), f32)

    def zb(i, c):
        z0[pl.ds(i * 16, 16)] = zeros16
        z1[pl.ds(i * 16, 16)] = zeros16
        return c

    lax.fori_loop(0, _NCOL // 16, zb, 0)

    bufs = (z0, z1)
    sems = (sem0, sem1)

    def body(i, c):
        for k in range(2):
            r = 2 * i + k
            g = base + r
            off = lax.rem(g, SEQ_LEN) * ENC_IN
            zk = bufs[k]
            sk = sems[k]

            @pl.when(i > 0)
            def _():
                pltpu.make_async_copy(zk, out_hbm.at[g - 2], sk).wait()
                poff = lax.rem(g - 2, SEQ_LEN) * ENC_IN
                zk[pl.ds(poff, 16)] = zeros16
                zk[pl.ds(poff + 16, 16)] = zeros16

            zk[pl.ds(off, 16)] = maskv[2 * r]
            zk[pl.ds(off + 16, 16)] = maskv[2 * r + 1]
            pltpu.async_copy(zk, out_hbm.at[g], sk)
        return c

    lax.fori_loop(0, _RPW // 2, body, 0)

    for k in range(2):
        g = base + _RPW - 2 + k
        pltpu.make_async_copy(bufs[k], out_hbm.at[g], sems[k]).wait()


@functools.partial(jax.jit, static_argnames=())
def kernel(x, mask, W_time, b_time, W_freq, b_freq, W_stat, b_stat,
           W_temp, b_temp, W_fhe, b_fhe, W_cross, b_cross):
    f32 = jnp.float32
    f_idx = jnp.arange(FPAD, dtype=f32)[:, None]
    t_idx = jnp.arange(SEQ_LEN, dtype=f32)[None, :]
    ang = (2.0 * jnp.pi / SEQ_LEN) * f_idx * t_idx
    live = (f_idx < N_BINS).astype(f32)
    cos_m = jnp.cos(ang) * live
    sin_m = jnp.sin(ang) * live

    b_time2 = b_time.reshape(1, D_MODEL)
    b_freq2 = b_freq.reshape(1, D_MODEL)
    b_stat2 = b_stat.reshape(1, D_MODEL)
    b_temp2 = b_temp.reshape(1, D_MODEL)
    b_fhe2 = b_fhe.reshape(1, D_MODEL)
    b_cross2 = b_cross.reshape(1, D_MODEL)

    full = lambda shape: pl.BlockSpec(shape, lambda b: (0,) * len(shape))
    per_b2 = pl.BlockSpec((1, SEQ_LEN, ENC_IN), lambda b: (b, 0, 0))

    inc_flat = pl.kernel(
        _inc_sc_body,
        out_type=jax.ShapeDtypeStruct((_ROWS, _NCOL), f32),
        mesh=plsc.VectorSubcoreMesh(core_axis_name="c", subcore_axis_name="s"),
        scratch_types=[
            pltpu.VMEM((2 * _RPW, 16), f32),
            pltpu.VMEM((_NCOL,), f32),
            pltpu.VMEM((_NCOL,), f32),
            pltpu.SemaphoreType.DMA,
            pltpu.SemaphoreType.DMA,
        ],
    )(mask.reshape(2 * _ROWS, 16))
    incidence = inc_flat.reshape(BS, SEQ_LEN, SEQ_LEN * ENC_IN)

    tn, fn, sn, the, fhe, cross = pl.pallas_call(
        _main_body,
        grid=(BS,),
        in_specs=[
            per_b2, per_b2,
            full((FPAD, SEQ_LEN)), full((FPAD, SEQ_LEN)),
            full((1, D_MODEL)), full((1, D_MODEL)),
            full((2, D_MODEL)), full((1, D_MODEL)),
            full((3, D_MODEL)), full((1, D_MODEL)),
            full((D_MODEL, D_MODEL)), full((1, D_MODEL)),
            full((D_MODEL, D_MODEL)), full((1, D_MODEL)),
            full((D_MODEL, D_MODEL)), full((1, D_MODEL)),
        ],
        out_specs=[
            pl.BlockSpec((1, SEQ_LEN, ENC_IN, D_MODEL), lambda b: (b, 0, 0, 0)),
            pl.BlockSpec((1, N_FREQ_MODES, ENC_IN, D_MODEL),
                         lambda b: (b, 0, 0, 0)),
            pl.BlockSpec((1, 3, ENC_IN, D_MODEL), lambda b: (b, 0, 0, 0)),
            pl.BlockSpec((1, SEQ_LEN, D_MODEL), lambda b: (b, 0, 0)),
            pl.BlockSpec((1, N_FREQ_MODES, D_MODEL), lambda b: (b, 0, 0)),
            pl.BlockSpec((1, 3, D_MODEL), lambda b: (b, 0, 0)),
        ],
        out_shape=[
            jax.ShapeDtypeStruct((BS, SEQ_LEN, ENC_IN, D_MODEL), f32),
            jax.ShapeDtypeStruct((BS, N_FREQ_MODES, ENC_IN, D_MODEL), f32),
            jax.ShapeDtypeStruct((BS, 3, ENC_IN, D_MODEL), f32),
            jax.ShapeDtypeStruct((BS, SEQ_LEN, D_MODEL), f32),
            jax.ShapeDtypeStruct((BS, N_FREQ_MODES, D_MODEL), f32),
            jax.ShapeDtypeStruct((BS, 3, D_MODEL), f32),
        ],
    )(x, mask, cos_m, sin_m, W_time, b_time2, W_freq, b_freq2, W_stat,
      b_stat2, W_temp, b_temp2, W_fhe, b_fhe2, W_cross, b_cross2)

    return (tn, fn, sn, the, fhe, cross, incidence, mask)
